# baseline (speedup 1.0000x reference)
import jax
import jax.numpy as jnp
from jax import lax
from jax.experimental import pallas as pl
from jax.experimental.pallas import tpu as pltpu
from jax.experimental.pallas import tpu_sc as plsc

N_NODES = 100000
N_EDGES = 6400000

_NW = 32
_EPW = N_EDGES // _NW
_BKT = 3136
_CAP = 7168
_NPAD = _NW * _NW * _CAP
_EPWG = _NPAD // _NW
_BINW = 1600
_GW = 512
_SEGW = 128
_AGGR = _NW * _BKT


def _sc_mesh():
    return plsc.VectorSubcoreMesh(core_axis_name="c", subcore_axis_name="s")


def _bin_body(src_hbm, dst_hbm, e0_hbm, e1_hbm,
              sbin, dbin, e0b, e1b, cnts_hbm,
              srcw, dstw, e0w, e1w, posw, fillv, cntvm, sem):
    s = lax.axis_index("s")
    w = s * 2 + lax.axis_index("c")
    base = w * _EPW
    rbase = w * 32 * _CAP

    for b in range(32):
        val = (b + 1) * _BKT

        def fill16(g, _, val=val):
            fillv[pl.ds(g * 16, 16)] = jnp.full((16,), val, jnp.int32)
            return 0

        lax.fori_loop(0, _CAP // 16, fill16, 0)
        pltpu.sync_copy(fillv, dbin.at[pl.ds(pl.multiple_of(rbase + b * _CAP, 128), _CAP)])

    iota = lax.iota(jnp.int32, 16)
    zero16 = jnp.zeros((16,), jnp.int32)
    cntvm[pl.ds(0, 16)] = zero16
    cntvm[pl.ds(16, 16)] = zero16

    def win(j, _):
        off = pl.multiple_of(base + j * _BINW, 64)
        pltpu.sync_copy(src_hbm.at[pl.ds(off, _BINW)], srcw)
        pltpu.sync_copy(dst_hbm.at[pl.ds(off, _BINW)], dstw)
        pltpu.sync_copy(e0_hbm.at[pl.ds(off, _BINW)], e0w)
        pltpu.sync_copy(e1_hbm.at[pl.ds(off, _BINW)], e1w)

        def vreg(g, _c):
            d = dstw[pl.ds(g * 16, 16)]
            q = lax.shift_right_logical(d, 6)
            bkt = lax.shift_right_logical(q * 1338, 16)
            ranks = jnp.zeros((16,), jnp.int32)
            nlater = jnp.zeros((16,), jnp.int32)
            one16 = jnp.full((16,), 1, jnp.int32)
            for lp in range(16):
                m = bkt == bkt[lp]
                e = jnp.where(m, one16, zero16)
                ranks = ranks + jnp.where(iota > lp, e, zero16)
                nlater = nlater + jnp.where(iota < lp, e, zero16)
            old = plsc.load_gather(cntvm, [bkt])
            is_last = nlater == 0
            plsc.store_scatter(cntvm, [bkt], old + ranks + 1, mask=is_last)
            pos = rbase + bkt * _CAP + old + ranks
            pos = jnp.minimum(pos, rbase + bkt * _CAP + (_CAP - 1))
            posw[g // 4, pl.ds((g % 4) * 16, 16)] = pos
            return 0

        lax.fori_loop(0, _BINW // 16, vreg, 0)

        for arr_w, arr_h in ((srcw, sbin), (dstw, dbin), (e0w, e0b),
                             (e1w, e1b)):
            hs = []
            for qc in range(_BINW // 64):
                hs.append(pltpu.async_copy(
                    arr_w.at[pl.ds(qc * 64, 64)],
                    arr_h.at[posw.at[qc]], sem))
            for h in hs:
                h.wait()
        return 0

    lax.fori_loop(0, _EPW // _BINW, win, 0)

    pltpu.sync_copy(cntvm, cnts_hbm.at[pl.ds(pl.multiple_of(w * 32, 32), 32)])


def _sc_bin(src, dst, e0, e1):
    fn = pl.kernel(
        _bin_body,
        out_type=[
            jax.ShapeDtypeStruct((_NPAD,), jnp.int32),
            jax.ShapeDtypeStruct((_NPAD,), jnp.int32),
            jax.ShapeDtypeStruct((_NPAD,), jnp.float32),
            jax.ShapeDtypeStruct((_NPAD,), jnp.float32),
            jax.ShapeDtypeStruct((1024,), jnp.int32),
        ],
        mesh=_sc_mesh(),
        compiler_params=pltpu.CompilerParams(needs_layout_passes=False),
        scratch_types=[
            pltpu.VMEM((_BINW,), jnp.int32),
            pltpu.VMEM((_BINW,), jnp.int32),
            pltpu.VMEM((_BINW,), jnp.float32),
            pltpu.VMEM((_BINW,), jnp.float32),
            pltpu.VMEM((_BINW // 64, 64), jnp.int32),
            pltpu.VMEM((_CAP,), jnp.int32),
            pltpu.VMEM((32,), jnp.int32),
            pltpu.SemaphoreType.DMA,
        ],
    )
    return fn(src, dst, e0, e1)


def _gather_body(table_hbm, src_hbm, e0_hbm, e1_hbm, wae_hbm, out_hbm,
                 srcw, e0w, e1w, bidx_v, rows_v, h_v, wae_v, sem):
    s = lax.axis_index("s")
    wid = s * 2 + lax.axis_index("c")
    base = wid * _EPWG

    pltpu.sync_copy(wae_hbm, wae_v)
    w12 = wae_v[pl.ds(0, 16)]
    w13 = wae_v[pl.ds(16, 16)]

    def step(j, _):
        off = pl.multiple_of(base + j * _GW, 64)
        pltpu.sync_copy(src_hbm.at[pl.ds(off, _GW)], srcw)
        pltpu.sync_copy(e0_hbm.at[pl.ds(off, _GW)], e0w)
        pltpu.sync_copy(e1_hbm.at[pl.ds(off, _GW)], e1w)

        def mk_bidx(g, _c):
            v = srcw[pl.ds(g * 16, 16)]
            bi = jnp.minimum(lax.shift_right_logical(v, 3), 12499)
            bidx_v[g // 8, pl.ds((g % 8) * 16, 16)] = bi
            return 0

        lax.fori_loop(0, _GW // 16, mk_bidx, 0)
        hs = []
        for q in range(_GW // 128):
            hs.append(pltpu.async_copy(
                table_hbm.at[bidx_v.at[q]],
                rows_v.at[pl.ds(q * 128, 128)], sem))
        for h in hs:
            h.wait()

        def extract(g, _c):
            v = srcw[pl.ds(g * 16, 16)]
            sub = lax.shift_left(lax.bitwise_and(v, 7), 4)
            e0v = e0w[pl.ds(g * 16, 16)]
            e1v = e1w[pl.ds(g * 16, 16)]
            for k in range(16):
                e = g * 16 + k
                grow = rows_v[e, pl.ds(sub[k], 16)]
                hrow = jnp.maximum(grow + e0v[k] * w12 + e1v[k] * w13, 0.0)
                h_v[e // 8, pl.ds((e % 8) * 16, 16)] = hrow
            return 0

        lax.fori_loop(0, _GW // 16, extract, 0)
        pltpu.sync_copy(h_v, out_hbm.at[pl.ds(pl.multiple_of(off // 8, 64), _GW // 8)])
        return 0

    lax.fori_loop(0, _EPWG // _GW, step, 0)


def _sc_gather(table_wide, sbin, e0b, e1b, wae):
    fn = pl.kernel(
        _gather_body,
        out_type=jax.ShapeDtypeStruct((_NPAD // 8, 128), jnp.float32),
        mesh=_sc_mesh(),
        compiler_params=pltpu.CompilerParams(needs_layout_passes=False),
        scratch_types=[
            pltpu.VMEM((_GW,), jnp.int32),
            pltpu.VMEM((_GW,), jnp.float32),
            pltpu.VMEM((_GW,), jnp.float32),
            pltpu.VMEM((_GW // 128, 128), jnp.int32),
            pltpu.VMEM((_GW, 128), jnp.float32),
            pltpu.VMEM((_GW // 8, 128), jnp.float32),
            pltpu.VMEM((32,), jnp.float32),
            pltpu.SemaphoreType.DMA,
        ],
    )
    return fn(table_wide, sbin, e0b, e1b, wae)


def _segmax_body(msg_hbm, dbin_hbm, cnts_hbm, agg_hbm, slab, msgw, dstw,
                 cntv, sem):
    s = lax.axis_index("s")
    t = s * 2 + lax.axis_index("c")
    tb = t * _BKT
    zero = jnp.zeros((16,), jnp.float32)

    def zr(r, _):
        for q in range(8):
            slab[r, pl.ds(q * 16, 16)] = zero
        return 0

    lax.fori_loop(0, _BKT // 4 + 1, zr, 0)

    pltpu.sync_copy(cnts_hbm, cntv)
    iota = lax.iota(jnp.int32, 16)
    c0 = plsc.load_gather(cntv, [iota * 32 + t])
    c1 = plsc.load_gather(cntv, [(iota + 16) * 32 + t])

    for w in range(32):
        n = c0[w] if w < 16 else c1[w - 16]
        region = (w * 32 + t) * _CAP
        trips = (n + (_SEGW - 1)) // _SEGW

        def win(j, _):
            eoff = pl.multiple_of(region + j * _SEGW, 128)
            pltpu.sync_copy(dbin_hbm.at[pl.ds(eoff, _SEGW)], dstw)
            pltpu.sync_copy(msg_hbm.at[pl.ds(pl.multiple_of(eoff // 8, 16), _SEGW // 8)], msgw)

            def grp(g, _c):
                loc = dstw[pl.ds(g * 16, 16)] - tb
                for k in range(16):
                    e = g * 16 + k
                    l = loc[k]
                    m0 = msgw[e // 8, pl.ds((e % 8) * 32, 16)]
                    m1 = msgw[e // 8, pl.ds((e % 8) * 32 + 16, 16)]
                    r = l // 4
                    cst = (l % 4) * 32
                    slab[r, pl.ds(cst, 16)] = jnp.maximum(
                        slab[r, pl.ds(cst, 16)], m0)
                    slab[r, pl.ds(cst + 16, 16)] = jnp.maximum(
                        slab[r, pl.ds(cst + 16, 16)], m1)
                return 0

            lax.fori_loop(0, _SEGW // 16, grp, 0)
            return 0

        lax.fori_loop(0, trips, win, 0)

    pltpu.sync_copy(slab.at[pl.ds(0, _BKT // 4)],
                    agg_hbm.at[pl.ds(pl.multiple_of(t * (_BKT // 4), 16), _BKT // 4)])


def _sc_segmax(msg8, dbin, cnts):
    fn = pl.kernel(
        _segmax_body,
        out_type=jax.ShapeDtypeStruct((_AGGR // 4, 128), jnp.float32),
        mesh=_sc_mesh(),
        compiler_params=pltpu.CompilerParams(needs_layout_passes=False),
        scratch_types=[
            pltpu.VMEM((_BKT // 4 + 1, 128), jnp.float32),
            pltpu.VMEM((_SEGW // 8, 256), jnp.float32),
            pltpu.VMEM((_SEGW,), jnp.int32),
            pltpu.VMEM((1024,), jnp.int32),
            pltpu.SemaphoreType.DMA,
        ],
    )
    return fn(msg8, dbin, cnts)


_EBLK = 256
_NODE_BLK = 4000


def _edge_body(h_ref, wbig_ref, bbig_ref, msg_ref):
    m = jax.lax.dot_general(h_ref[...], wbig_ref[...], (((1,), (0,)), ((), ())),
                            preferred_element_type=jnp.float32)
    msg_ref[...] = jnp.maximum(m + bbig_ref[...], 0.0)


def _edge_mlp(h8, wbig, bbig):
    nblk = (_NPAD // 8) // _EBLK
    return pl.pallas_call(
        _edge_body,
        grid=(nblk,),
        in_specs=[
            pl.BlockSpec((_EBLK, 128), lambda i: (i, 0)),
            pl.BlockSpec((128, 256), lambda i: (0, 0)),
            pl.BlockSpec((1, 256), lambda i: (0, 0)),
        ],
        out_specs=pl.BlockSpec((_EBLK, 256), lambda i: (i, 0)),
        out_shape=jax.ShapeDtypeStruct((_NPAD // 8, 256), jnp.float32),
    )(h8, wbig, bbig)


def _node_body(x_ref, agg_ref, w2ax_ref, w2aa_ref, b2a_ref, w2b_ref, b2b_ref,
               w1a_ref, b1a_ref, comb_ref, np_ref):
    x = x_ref[...]
    agg = agg_ref[...]
    h = jax.lax.dot_general(x, w2ax_ref[...], (((1,), (0,)), ((), ())),
                            preferred_element_type=jnp.float32)
    h = h + jax.lax.dot_general(agg, w2aa_ref[...], (((1,), (0,)), ((), ())),
                                preferred_element_type=jnp.float32)
    h = jnp.maximum(h + b2a_ref[...], 0.0)
    comb = jax.lax.dot_general(h, w2b_ref[...], (((1,), (0,)), ((), ())),
                               preferred_element_type=jnp.float32)
    comb = jnp.maximum(comb + b2b_ref[...], 0.0)
    comb_ref[...] = comb
    x_next = jnp.concatenate([x[:, :11], comb], axis=1)
    npre = jax.lax.dot_general(x_next, w1a_ref[...], (((1,), (0,)), ((), ())),
                               preferred_element_type=jnp.float32)
    np_ref[...] = npre + b1a_ref[...]


def _node_mlp(x, agg, w2ax, w2aa, b2a, w2b, b2b, w1a, b1a):
    nblk = N_NODES // _NODE_BLK
    return pl.pallas_call(
        _node_body,
        grid=(nblk,),
        in_specs=[
            pl.BlockSpec((_NODE_BLK, 12), lambda i: (i, 0)),
            pl.BlockSpec((_NODE_BLK, 32), lambda i: (i, 0)),
            pl.BlockSpec((12, 16), lambda i: (0, 0)),
            pl.BlockSpec((32, 16), lambda i: (0, 0)),
            pl.BlockSpec((1, 16), lambda i: (0, 0)),
            pl.BlockSpec((16, 1), lambda i: (0, 0)),
            pl.BlockSpec((1, 1), lambda i: (0, 0)),
            pl.BlockSpec((12, 16), lambda i: (0, 0)),
            pl.BlockSpec((1, 16), lambda i: (0, 0)),
        ],
        out_specs=[
            pl.BlockSpec((_NODE_BLK, 1), lambda i: (i, 0)),
            pl.BlockSpec((_NODE_BLK, 16), lambda i: (i, 0)),
        ],
        out_shape=[
            jax.ShapeDtypeStruct((N_NODES, 1), jnp.float32),
            jax.ShapeDtypeStruct((N_NODES, 16), jnp.float32),
        ],
    )(x, agg, w2ax, w2aa, b2a, w2b, b2b, w1a, b1a)


def _pre_body(x_ref, w1a_ref, b1a_ref, np_ref):
    npre = jax.lax.dot_general(x_ref[...], w1a_ref[...], (((1,), (0,)), ((), ())),
                               preferred_element_type=jnp.float32)
    np_ref[...] = npre + b1a_ref[...]


def _node_pre(x, w1a, b1a):
    nblk = N_NODES // _NODE_BLK
    return pl.pallas_call(
        _pre_body,
        grid=(nblk,),
        in_specs=[
            pl.BlockSpec((_NODE_BLK, 12), lambda i: (i, 0)),
            pl.BlockSpec((12, 16), lambda i: (0, 0)),
            pl.BlockSpec((1, 16), lambda i: (0, 0)),
        ],
        out_specs=pl.BlockSpec((_NODE_BLK, 16), lambda i: (i, 0)),
        out_shape=jax.ShapeDtypeStruct((N_NODES, 16), jnp.float32),
    )(x, w1a, b1a)


def kernel(x, edge_index, edge_attr, W1a, b1a, W1b, b1b, W2a, b2a, W2b, b2b):
    src = edge_index[0].astype(jnp.int32)
    dst = edge_index[1].astype(jnp.int32)
    e0 = edge_attr[:, 0]
    e1 = edge_attr[:, 1]

    w1a_x = W1a[:12]
    wae = jnp.reshape(W1a[12:14], (32,))
    b1a_r = b1a.reshape(1, 16)
    wbig = jnp.kron(jnp.eye(8, dtype=jnp.float32), W1b)
    bbig = jnp.tile(b1b, 8).reshape(1, 256)
    w2a_x = W2a[:12]
    w2a_a = W2a[12:44]
    b2a_r = b2a.reshape(1, 16)
    b2b_r = b2b.reshape(1, 1)

    sbin, dbin, e0b, e1b, cnts = _sc_bin(src, dst, e0, e1)

    xc = x[:, :11]
    npre = _node_pre(x, w1a_x, b1a_r)
    x_cur = x
    for _ in range(3):
        h8 = _sc_gather(jnp.reshape(npre, (12500, 128)), sbin, e0b, e1b, wae)
        msg8 = _edge_mlp(h8, wbig, bbig)
        agg8 = _sc_segmax(msg8, dbin, cnts)
        agg = jnp.reshape(agg8[:N_NODES // 4], (N_NODES, 32))
        comb, npre = _node_mlp(x_cur, agg, w2a_x, w2a_a, b2a_r, W2b,
                               b2b_r, w1a_x, b1a_r)
        x_cur = jnp.concatenate([xc, comb], axis=1)
    return x_cur


# speedup vs baseline: 1.0051x; 1.0051x over previous
import jax
import jax.numpy as jnp
from jax import lax
from jax.experimental import pallas as pl
from jax.experimental.pallas import tpu as pltpu
from jax.experimental.pallas import tpu_sc as plsc

N_NODES = 100000
N_EDGES = 6400000

_NW = 32
_EPW = N_EDGES // _NW
_BKT = 3136
_CAP = 7168
_NPAD = _NW * _NW * _CAP
_EPWG = _NPAD // _NW
_BINW = 1600
_GW = 512
_SEGW = 128
_AGGR = _NW * _BKT


def _sc_mesh():
    return plsc.VectorSubcoreMesh(core_axis_name="c", subcore_axis_name="s")


def _bin_body(src_hbm, dst_hbm, e0_hbm, e1_hbm,
              sbin, dbin, e0b, e1b, cnts_hbm,
              srcw, dstw, e0w, e1w, posw, fillv, cntvm, sem):
    s = lax.axis_index("s")
    w = s * 2 + lax.axis_index("c")
    base = w * _EPW
    rbase = w * 32 * _CAP

    for b in range(32):
        val = (b + 1) * _BKT

        def fill16(g, _, val=val):
            fillv[pl.ds(g * 16, 16)] = jnp.full((16,), val, jnp.int32)
            return 0

        lax.fori_loop(0, _CAP // 16, fill16, 0)
        pltpu.sync_copy(fillv, dbin.at[pl.ds(pl.multiple_of(rbase + b * _CAP, 128), _CAP)])

    iota = lax.iota(jnp.int32, 16)
    zero16 = jnp.zeros((16,), jnp.int32)

    def win(j, carry):
        off = pl.multiple_of(base + j * _BINW, 64)
        pltpu.sync_copy(src_hbm.at[pl.ds(off, _BINW)], srcw)
        pltpu.sync_copy(dst_hbm.at[pl.ds(off, _BINW)], dstw)
        pltpu.sync_copy(e0_hbm.at[pl.ds(off, _BINW)], e0w)
        pltpu.sync_copy(e1_hbm.at[pl.ds(off, _BINW)], e1w)

        def vreg(g, cnts01):
            cnt0, cnt1 = cnts01
            d = dstw[pl.ds(g * 16, 16)]
            q = lax.shift_right_logical(d, 6)
            bkt = lax.shift_right_logical(q * 1338, 16)
            ranks = jnp.zeros((16,), jnp.int32)
            one16 = jnp.full((16,), 1, jnp.int32)
            for lp in range(16):
                m = bkt == bkt[lp]
                e = jnp.where(m, one16, zero16)
                ranks = ranks + jnp.where(iota > lp, e, zero16)
            old = jnp.zeros((16,), jnp.int32)
            for b in range(16):
                old = jnp.where(bkt == b, cnt0[b], old)
            for b in range(16):
                old = jnp.where(bkt == (b + 16), cnt1[b], old)
            for lp in range(16):
                c = bkt[lp]
                v = old[lp] + ranks[lp] + 1
                cnt0 = jnp.where(iota == c, v, cnt0)
                cnt1 = jnp.where(iota == (c - 16), v, cnt1)
            pos = rbase + bkt * _CAP + old + ranks
            pos = jnp.minimum(pos, rbase + bkt * _CAP + (_CAP - 1))
            posw[g // 4, pl.ds((g % 4) * 16, 16)] = pos
            return (cnt0, cnt1)

        cnt0, cnt1 = lax.fori_loop(0, _BINW // 16, vreg, carry)

        for arr_w, arr_h in ((srcw, sbin), (dstw, dbin), (e0w, e0b),
                             (e1w, e1b)):
            hs = []
            for qc in range(_BINW // 64):
                hs.append(pltpu.async_copy(
                    arr_w.at[pl.ds(qc * 64, 64)],
                    arr_h.at[posw.at[qc]], sem))
            for h in hs:
                h.wait()
        return (cnt0, cnt1)

    z16 = jnp.zeros((16,), jnp.int32)
    fcnt0, fcnt1 = lax.fori_loop(0, _EPW // _BINW, win, (z16, z16))
    cntvm[pl.ds(0, 16)] = fcnt0
    cntvm[pl.ds(16, 16)] = fcnt1
    pltpu.sync_copy(cntvm, cnts_hbm.at[pl.ds(pl.multiple_of(w * 32, 32), 32)])


def _sc_bin(src, dst, e0, e1):
    fn = pl.kernel(
        _bin_body,
        out_type=[
            jax.ShapeDtypeStruct((_NPAD,), jnp.int32),
            jax.ShapeDtypeStruct((_NPAD,), jnp.int32),
            jax.ShapeDtypeStruct((_NPAD,), jnp.float32),
            jax.ShapeDtypeStruct((_NPAD,), jnp.float32),
            jax.ShapeDtypeStruct((1024,), jnp.int32),
        ],
        mesh=_sc_mesh(),
        scratch_types=[
            pltpu.VMEM((_BINW,), jnp.int32),
            pltpu.VMEM((_BINW,), jnp.int32),
            pltpu.VMEM((_BINW,), jnp.float32),
            pltpu.VMEM((_BINW,), jnp.float32),
            pltpu.VMEM((_BINW // 64, 64), jnp.int32),
            pltpu.VMEM((_CAP,), jnp.int32),
            pltpu.VMEM((32,), jnp.int32),
            pltpu.SemaphoreType.DMA,
        ],
    )
    return fn(src, dst, e0, e1)


def _gather_body(table_hbm, src_hbm, e0_hbm, e1_hbm, wae_hbm, out_hbm,
                 srcw, e0w, e1w, bidx_v, rows_v, h_v, wae_v, sem):
    s = lax.axis_index("s")
    wid = s * 2 + lax.axis_index("c")
    base = wid * _EPWG

    pltpu.sync_copy(wae_hbm, wae_v)
    w12 = wae_v[pl.ds(0, 16)]
    w13 = wae_v[pl.ds(16, 16)]

    def step(j, _):
        off = pl.multiple_of(base + j * _GW, 64)
        pltpu.sync_copy(src_hbm.at[pl.ds(off, _GW)], srcw)
        pltpu.sync_copy(e0_hbm.at[pl.ds(off, _GW)], e0w)
        pltpu.sync_copy(e1_hbm.at[pl.ds(off, _GW)], e1w)

        def mk_bidx(g, _c):
            v = srcw[pl.ds(g * 16, 16)]
            bi = jnp.minimum(lax.shift_right_logical(v, 3), 12499)
            bidx_v[g // 8, pl.ds((g % 8) * 16, 16)] = bi
            return 0

        lax.fori_loop(0, _GW // 16, mk_bidx, 0)
        hs = []
        for q in range(_GW // 128):
            hs.append(pltpu.async_copy(
                table_hbm.at[bidx_v.at[q]],
                rows_v.at[pl.ds(q * 128, 128)], sem))
        for h in hs:
            h.wait()

        def extract(g, _c):
            v = srcw[pl.ds(g * 16, 16)]
            sub = lax.shift_left(lax.bitwise_and(v, 7), 4)
            e0v = e0w[pl.ds(g * 16, 16)]
            e1v = e1w[pl.ds(g * 16, 16)]
            for k in range(16):
                e = g * 16 + k
                grow = rows_v[e, pl.ds(sub[k], 16)]
                hrow = jnp.maximum(grow + e0v[k] * w12 + e1v[k] * w13, 0.0)
                h_v[e // 8, pl.ds((e % 8) * 16, 16)] = hrow
            return 0

        lax.fori_loop(0, _GW // 16, extract, 0)
        pltpu.sync_copy(h_v, out_hbm.at[pl.ds(pl.multiple_of(off // 8, 64), _GW // 8)])
        return 0

    lax.fori_loop(0, _EPWG // _GW, step, 0)


def _sc_gather(table_wide, sbin, e0b, e1b, wae):
    fn = pl.kernel(
        _gather_body,
        out_type=jax.ShapeDtypeStruct((_NPAD // 8, 128), jnp.float32),
        mesh=_sc_mesh(),
        scratch_types=[
            pltpu.VMEM((_GW,), jnp.int32),
            pltpu.VMEM((_GW,), jnp.float32),
            pltpu.VMEM((_GW,), jnp.float32),
            pltpu.VMEM((_GW // 128, 128), jnp.int32),
            pltpu.VMEM((_GW, 128), jnp.float32),
            pltpu.VMEM((_GW // 8, 128), jnp.float32),
            pltpu.VMEM((32,), jnp.float32),
            pltpu.SemaphoreType.DMA,
        ],
    )
    return fn(table_wide, sbin, e0b, e1b, wae)


def _segmax_body(msg_hbm, dbin_hbm, cnts_hbm, agg_hbm, slab, msgw, dstw,
                 cntv, sem):
    s = lax.axis_index("s")
    t = s * 2 + lax.axis_index("c")
    tb = t * _BKT
    zero = jnp.zeros((16,), jnp.float32)

    def zr(r, _):
        for q in range(8):
            slab[r, pl.ds(q * 16, 16)] = zero
        return 0

    lax.fori_loop(0, _BKT // 4 + 1, zr, 0)

    pltpu.sync_copy(cnts_hbm, cntv)
    iota = lax.iota(jnp.int32, 16)
    c0 = plsc.load_gather(cntv, [iota * 32 + t])
    c1 = plsc.load_gather(cntv, [(iota + 16) * 32 + t])

    for w in range(32):
        n = c0[w] if w < 16 else c1[w - 16]
        region = (w * 32 + t) * _CAP
        trips = (n + (_SEGW - 1)) // _SEGW

        def win(j, _):
            eoff = pl.multiple_of(region + j * _SEGW, 128)
            pltpu.sync_copy(dbin_hbm.at[pl.ds(eoff, _SEGW)], dstw)
            pltpu.sync_copy(msg_hbm.at[pl.ds(pl.multiple_of(eoff // 8, 16), _SEGW // 8)], msgw)

            def grp(g, _c):
                loc = dstw[pl.ds(g * 16, 16)] - tb
                for k in range(16):
                    e = g * 16 + k
                    l = loc[k]
                    m0 = msgw[e // 8, pl.ds((e % 8) * 32, 16)]
                    m1 = msgw[e // 8, pl.ds((e % 8) * 32 + 16, 16)]
                    r = l // 4
                    cst = (l % 4) * 32
                    slab[r, pl.ds(cst, 16)] = jnp.maximum(
                        slab[r, pl.ds(cst, 16)], m0)
                    slab[r, pl.ds(cst + 16, 16)] = jnp.maximum(
                        slab[r, pl.ds(cst + 16, 16)], m1)
                return 0

            lax.fori_loop(0, _SEGW // 16, grp, 0)
            return 0

        lax.fori_loop(0, trips, win, 0)

    pltpu.sync_copy(slab.at[pl.ds(0, _BKT // 4)],
                    agg_hbm.at[pl.ds(pl.multiple_of(t * (_BKT // 4), 16), _BKT // 4)])


def _sc_segmax(msg8, dbin, cnts):
    fn = pl.kernel(
        _segmax_body,
        out_type=jax.ShapeDtypeStruct((_AGGR // 4, 128), jnp.float32),
        mesh=_sc_mesh(),
        compiler_params=pltpu.CompilerParams(needs_layout_passes=False),
        scratch_types=[
            pltpu.VMEM((_BKT // 4 + 1, 128), jnp.float32),
            pltpu.VMEM((_SEGW // 8, 256), jnp.float32),
            pltpu.VMEM((_SEGW,), jnp.int32),
            pltpu.VMEM((1024,), jnp.int32),
            pltpu.SemaphoreType.DMA,
        ],
    )
    return fn(msg8, dbin, cnts)


_EBLK = 256
_NODE_BLK = 4000


def _edge_body(h_ref, wbig_ref, bbig_ref, msg_ref):
    m = jax.lax.dot_general(h_ref[...], wbig_ref[...], (((1,), (0,)), ((), ())),
                            preferred_element_type=jnp.float32)
    msg_ref[...] = jnp.maximum(m + bbig_ref[...], 0.0)


def _edge_mlp(h8, wbig, bbig):
    nblk = (_NPAD // 8) // _EBLK
    return pl.pallas_call(
        _edge_body,
        grid=(nblk,),
        in_specs=[
            pl.BlockSpec((_EBLK, 128), lambda i: (i, 0)),
            pl.BlockSpec((128, 256), lambda i: (0, 0)),
            pl.BlockSpec((1, 256), lambda i: (0, 0)),
        ],
        out_specs=pl.BlockSpec((_EBLK, 256), lambda i: (i, 0)),
        out_shape=jax.ShapeDtypeStruct((_NPAD // 8, 256), jnp.float32),
    )(h8, wbig, bbig)


def _node_body(x_ref, agg_ref, w2ax_ref, w2aa_ref, b2a_ref, w2b_ref, b2b_ref,
               w1a_ref, b1a_ref, comb_ref, np_ref):
    x = x_ref[...]
    agg = agg_ref[...]
    h = jax.lax.dot_general(x, w2ax_ref[...], (((1,), (0,)), ((), ())),
                            preferred_element_type=jnp.float32)
    h = h + jax.lax.dot_general(agg, w2aa_ref[...], (((1,), (0,)), ((), ())),
                                preferred_element_type=jnp.float32)
    h = jnp.maximum(h + b2a_ref[...], 0.0)
    comb = jax.lax.dot_general(h, w2b_ref[...], (((1,), (0,)), ((), ())),
                               preferred_element_type=jnp.float32)
    comb = jnp.maximum(comb + b2b_ref[...], 0.0)
    comb_ref[...] = comb
    x_next = jnp.concatenate([x[:, :11], comb], axis=1)
    npre = jax.lax.dot_general(x_next, w1a_ref[...], (((1,), (0,)), ((), ())),
                               preferred_element_type=jnp.float32)
    np_ref[...] = npre + b1a_ref[...]


def _node_mlp(x, agg, w2ax, w2aa, b2a, w2b, b2b, w1a, b1a):
    nblk = N_NODES // _NODE_BLK
    return pl.pallas_call(
        _node_body,
        grid=(nblk,),
        in_specs=[
            pl.BlockSpec((_NODE_BLK, 12), lambda i: (i, 0)),
            pl.BlockSpec((_NODE_BLK, 32), lambda i: (i, 0)),
            pl.BlockSpec((12, 16), lambda i: (0, 0)),
            pl.BlockSpec((32, 16), lambda i: (0, 0)),
            pl.BlockSpec((1, 16), lambda i: (0, 0)),
            pl.BlockSpec((16, 1), lambda i: (0, 0)),
            pl.BlockSpec((1, 1), lambda i: (0, 0)),
            pl.BlockSpec((12, 16), lambda i: (0, 0)),
            pl.BlockSpec((1, 16), lambda i: (0, 0)),
        ],
        out_specs=[
            pl.BlockSpec((_NODE_BLK, 1), lambda i: (i, 0)),
            pl.BlockSpec((_NODE_BLK, 16), lambda i: (i, 0)),
        ],
        out_shape=[
            jax.ShapeDtypeStruct((N_NODES, 1), jnp.float32),
            jax.ShapeDtypeStruct((N_NODES, 16), jnp.float32),
        ],
    )(x, agg, w2ax, w2aa, b2a, w2b, b2b, w1a, b1a)


def _pre_body(x_ref, w1a_ref, b1a_ref, np_ref):
    npre = jax.lax.dot_general(x_ref[...], w1a_ref[...], (((1,), (0,)), ((), ())),
                               preferred_element_type=jnp.float32)
    np_ref[...] = npre + b1a_ref[...]


def _node_pre(x, w1a, b1a):
    nblk = N_NODES // _NODE_BLK
    return pl.pallas_call(
        _pre_body,
        grid=(nblk,),
        in_specs=[
            pl.BlockSpec((_NODE_BLK, 12), lambda i: (i, 0)),
            pl.BlockSpec((12, 16), lambda i: (0, 0)),
            pl.BlockSpec((1, 16), lambda i: (0, 0)),
        ],
        out_specs=pl.BlockSpec((_NODE_BLK, 16), lambda i: (i, 0)),
        out_shape=jax.ShapeDtypeStruct((N_NODES, 16), jnp.float32),
    )(x, w1a, b1a)


def kernel(x, edge_index, edge_attr, W1a, b1a, W1b, b1b, W2a, b2a, W2b, b2b):
    src = edge_index[0].astype(jnp.int32)
    dst = edge_index[1].astype(jnp.int32)
    e0 = edge_attr[:, 0]
    e1 = edge_attr[:, 1]

    w1a_x = W1a[:12]
    wae = jnp.reshape(W1a[12:14], (32,))
    b1a_r = b1a.reshape(1, 16)
    wbig = jnp.kron(jnp.eye(8, dtype=jnp.float32), W1b)
    bbig = jnp.tile(b1b, 8).reshape(1, 256)
    w2a_x = W2a[:12]
    w2a_a = W2a[12:44]
    b2a_r = b2a.reshape(1, 16)
    b2b_r = b2b.reshape(1, 1)

    sbin, dbin, e0b, e1b, cnts = _sc_bin(src, dst, e0, e1)

    xc = x[:, :11]
    npre = _node_pre(x, w1a_x, b1a_r)
    x_cur = x
    for _ in range(3):
        h8 = _sc_gather(jnp.reshape(npre, (12500, 128)), sbin, e0b, e1b, wae)
        msg8 = _edge_mlp(h8, wbig, bbig)
        agg8 = _sc_segmax(msg8, dbin, cnts)
        agg = jnp.reshape(agg8[:N_NODES // 4], (N_NODES, 32))
        comb, npre = _node_mlp(x_cur, agg, w2a_x, w2a_a, b2a_r, W2b,
                               b2b_r, w1a_x, b1a_r)
        x_cur = jnp.concatenate([xc, comb], axis=1)
    return x_cur


# speedup vs baseline: 2.6503x; 2.6370x over previous
import jax
import jax.numpy as jnp
from jax import lax
from jax.experimental import pallas as pl
from jax.experimental.pallas import tpu as pltpu
from jax.experimental.pallas import tpu_sc as plsc

N_NODES = 100000
N_EDGES = 6400000

_NW = 32
_EPW = N_EDGES // _NW
_BKT = 3136
_CAP = 7168
_NPAD = _NW * _NW * _CAP
_EPWG = _NPAD // _NW
_BINW = 1600
_GW = 512
_SEGW = 128
_AGGR = _NW * _BKT


def _sc_mesh():
    return plsc.VectorSubcoreMesh(core_axis_name="c", subcore_axis_name="s")


def _bin_body(src_hbm, dst_hbm, e0_hbm, e1_hbm,
              sbin, dbin, e0b, e1b, cnts_hbm,
              srcw, dstw, e0w, e1w, posw, fillv, cntvm, sem):
    s = lax.axis_index("s")
    w = s * 2 + lax.axis_index("c")
    base = w * _EPW
    rbase = w * 32 * _CAP

    for b in range(32):
        val = (b + 1) * _BKT

        def fill16(g, _, val=val):
            fillv[pl.ds(g * 16, 16)] = jnp.full((16,), val, jnp.int32)
            return 0

        lax.fori_loop(0, _CAP // 16, fill16, 0)
        pltpu.sync_copy(fillv, dbin.at[pl.ds(pl.multiple_of(rbase + b * _CAP, 128), _CAP)])

    def fillsp(g, _):
        fillv[pl.ds(g * 16, 16)] = jnp.bitwise_and(
            (lax.iota(jnp.int32, 16) + g * 16) * 523, 32767)
        return 0

    lax.fori_loop(0, _CAP // 16, fillsp, 0)
    for b in range(32):
        pltpu.sync_copy(fillv, sbin.at[pl.ds(pl.multiple_of(rbase + b * _CAP, 128), _CAP)])

    iota = lax.iota(jnp.int32, 16)
    zero16 = jnp.zeros((16,), jnp.int32)

    def win(j, carry):
        off = pl.multiple_of(base + j * _BINW, 64)
        pltpu.sync_copy(src_hbm.at[pl.ds(off, _BINW)], srcw)
        pltpu.sync_copy(dst_hbm.at[pl.ds(off, _BINW)], dstw)
        pltpu.sync_copy(e0_hbm.at[pl.ds(off, _BINW)], e0w)
        pltpu.sync_copy(e1_hbm.at[pl.ds(off, _BINW)], e1w)

        def vreg(g, cnts01):
            cnt0, cnt1 = cnts01
            d = dstw[pl.ds(g * 16, 16)]
            q = lax.shift_right_logical(d, 6)
            bkt = lax.shift_right_logical(q * 1338, 16)
            ranks = jnp.zeros((16,), jnp.int32)
            one16 = jnp.full((16,), 1, jnp.int32)
            for lp in range(16):
                m = bkt == bkt[lp]
                e = jnp.where(m, one16, zero16)
                ranks = ranks + jnp.where(iota > lp, e, zero16)
            old = jnp.zeros((16,), jnp.int32)
            for b in range(16):
                old = jnp.where(bkt == b, cnt0[b], old)
            for b in range(16):
                old = jnp.where(bkt == (b + 16), cnt1[b], old)
            for lp in range(16):
                c = bkt[lp]
                v = old[lp] + ranks[lp] + 1
                cnt0 = jnp.where(iota == c, v, cnt0)
                cnt1 = jnp.where(iota == (c - 16), v, cnt1)
            pos = rbase + bkt * _CAP + old + ranks
            pos = jnp.minimum(pos, rbase + bkt * _CAP + (_CAP - 1))
            posw[g // 4, pl.ds((g % 4) * 16, 16)] = pos
            return (cnt0, cnt1)

        cnt0, cnt1 = lax.fori_loop(0, _BINW // 16, vreg, carry)

        for arr_w, arr_h in ((srcw, sbin), (dstw, dbin), (e0w, e0b),
                             (e1w, e1b)):
            hs = []
            for qc in range(_BINW // 64):
                hs.append(pltpu.async_copy(
                    arr_w.at[pl.ds(qc * 64, 64)],
                    arr_h.at[posw.at[qc]], sem))
            for h in hs:
                h.wait()
        return (cnt0, cnt1)

    z16 = jnp.zeros((16,), jnp.int32)
    fcnt0, fcnt1 = lax.fori_loop(0, _EPW // _BINW, win, (z16, z16))
    cntvm[pl.ds(0, 16)] = fcnt0
    cntvm[pl.ds(16, 16)] = fcnt1
    pltpu.sync_copy(cntvm, cnts_hbm.at[pl.ds(pl.multiple_of(w * 32, 32), 32)])


def _sc_bin(src, dst, e0, e1):
    fn = pl.kernel(
        _bin_body,
        out_type=[
            jax.ShapeDtypeStruct((_NPAD,), jnp.int32),
            jax.ShapeDtypeStruct((_NPAD,), jnp.int32),
            jax.ShapeDtypeStruct((_NPAD,), jnp.float32),
            jax.ShapeDtypeStruct((_NPAD,), jnp.float32),
            jax.ShapeDtypeStruct((1024,), jnp.int32),
        ],
        mesh=_sc_mesh(),
        scratch_types=[
            pltpu.VMEM((_BINW,), jnp.int32),
            pltpu.VMEM((_BINW,), jnp.int32),
            pltpu.VMEM((_BINW,), jnp.float32),
            pltpu.VMEM((_BINW,), jnp.float32),
            pltpu.VMEM((_BINW // 64, 64), jnp.int32),
            pltpu.VMEM((_CAP,), jnp.int32),
            pltpu.VMEM((32,), jnp.int32),
            pltpu.SemaphoreType.DMA,
        ],
    )
    return fn(src, dst, e0, e1)


def _gather_body(table_hbm, src_hbm, e0_hbm, e1_hbm, wae_hbm, out_hbm,
                 srcw, e0w, e1w, bidx_v, rows_v, h_v, wae_v, sem):
    s = lax.axis_index("s")
    wid = s * 2 + lax.axis_index("c")
    base = wid * _EPWG

    pltpu.sync_copy(wae_hbm, wae_v)
    w12 = wae_v[pl.ds(0, 16)]
    w13 = wae_v[pl.ds(16, 16)]

    def step(j, _):
        off = pl.multiple_of(base + j * _GW, 64)
        pltpu.sync_copy(src_hbm.at[pl.ds(off, _GW)], srcw)
        pltpu.sync_copy(e0_hbm.at[pl.ds(off, _GW)], e0w)
        pltpu.sync_copy(e1_hbm.at[pl.ds(off, _GW)], e1w)

        def mk_bidx(g, _c):
            v = srcw[pl.ds(g * 16, 16)]
            bi = jnp.minimum(lax.shift_right_logical(v, 3), 12499)
            bidx_v[g // 8, pl.ds((g % 8) * 16, 16)] = bi
            return 0

        lax.fori_loop(0, _GW // 16, mk_bidx, 0)
        hs = []
        for q in range(_GW // 128):
            hs.append(pltpu.async_copy(
                table_hbm.at[bidx_v.at[q]],
                rows_v.at[pl.ds(q * 128, 128)], sem))
        for h in hs:
            h.wait()

        def extract(g, _c):
            v = srcw[pl.ds(g * 16, 16)]
            sub = lax.shift_left(lax.bitwise_and(v, 7), 4)
            e0v = e0w[pl.ds(g * 16, 16)]
            e1v = e1w[pl.ds(g * 16, 16)]
            for k in range(16):
                e = g * 16 + k
                grow = rows_v[e, pl.ds(sub[k], 16)]
                hrow = jnp.maximum(grow + e0v[k] * w12 + e1v[k] * w13, 0.0)
                h_v[e // 8, pl.ds((e % 8) * 16, 16)] = hrow
            return 0

        lax.fori_loop(0, _GW // 16, extract, 0)
        pltpu.sync_copy(h_v, out_hbm.at[pl.ds(pl.multiple_of(off // 8, 64), _GW // 8)])
        return 0

    lax.fori_loop(0, _EPWG // _GW, step, 0)


def _sc_gather(table_wide, sbin, e0b, e1b, wae):
    fn = pl.kernel(
        _gather_body,
        out_type=jax.ShapeDtypeStruct((_NPAD // 8, 128), jnp.float32),
        mesh=_sc_mesh(),
        scratch_types=[
            pltpu.VMEM((_GW,), jnp.int32),
            pltpu.VMEM((_GW,), jnp.float32),
            pltpu.VMEM((_GW,), jnp.float32),
            pltpu.VMEM((_GW // 128, 128), jnp.int32),
            pltpu.VMEM((_GW, 128), jnp.float32),
            pltpu.VMEM((_GW // 8, 128), jnp.float32),
            pltpu.VMEM((32,), jnp.float32),
            pltpu.SemaphoreType.DMA,
        ],
    )
    return fn(table_wide, sbin, e0b, e1b, wae)


def _segmax_body(msg_hbm, dbin_hbm, cnts_hbm, agg_hbm, slab, msgw, dstw,
                 cntv, sem):
    s = lax.axis_index("s")
    t = s * 2 + lax.axis_index("c")
    tb = t * _BKT
    zero = jnp.zeros((16,), jnp.float32)

    def zr(r, _):
        for q in range(8):
            slab[r, pl.ds(q * 16, 16)] = zero
        return 0

    lax.fori_loop(0, _BKT // 4 + 1, zr, 0)

    pltpu.sync_copy(cnts_hbm, cntv)
    iota = lax.iota(jnp.int32, 16)
    c0 = plsc.load_gather(cntv, [iota * 32 + t])
    c1 = plsc.load_gather(cntv, [(iota + 16) * 32 + t])

    for w in range(32):
        n = c0[w] if w < 16 else c1[w - 16]
        region = (w * 32 + t) * _CAP
        trips = (n + (_SEGW - 1)) // _SEGW

        def win(j, _):
            eoff = pl.multiple_of(region + j * _SEGW, 128)
            pltpu.sync_copy(dbin_hbm.at[pl.ds(eoff, _SEGW)], dstw)
            pltpu.sync_copy(msg_hbm.at[pl.ds(pl.multiple_of(eoff // 8, 16), _SEGW // 8)], msgw)

            def grp(g, _c):
                loc = dstw[pl.ds(g * 16, 16)] - tb
                for k in range(16):
                    e = g * 16 + k
                    l = loc[k]
                    m0 = msgw[e // 8, pl.ds((e % 8) * 32, 16)]
                    m1 = msgw[e // 8, pl.ds((e % 8) * 32 + 16, 16)]
                    r = l // 4
                    cst = (l % 4) * 32
                    slab[r, pl.ds(cst, 16)] = jnp.maximum(
                        slab[r, pl.ds(cst, 16)], m0)
                    slab[r, pl.ds(cst + 16, 16)] = jnp.maximum(
                        slab[r, pl.ds(cst + 16, 16)], m1)
                return 0

            lax.fori_loop(0, _SEGW // 16, grp, 0)
            return 0

        lax.fori_loop(0, trips, win, 0)

    pltpu.sync_copy(slab.at[pl.ds(0, _BKT // 4)],
                    agg_hbm.at[pl.ds(pl.multiple_of(t * (_BKT // 4), 16), _BKT // 4)])


def _sc_segmax(msg8, dbin, cnts):
    fn = pl.kernel(
        _segmax_body,
        out_type=jax.ShapeDtypeStruct((_AGGR // 4, 128), jnp.float32),
        mesh=_sc_mesh(),
        compiler_params=pltpu.CompilerParams(needs_layout_passes=False),
        scratch_types=[
            pltpu.VMEM((_BKT // 4 + 1, 128), jnp.float32),
            pltpu.VMEM((_SEGW // 8, 256), jnp.float32),
            pltpu.VMEM((_SEGW,), jnp.int32),
            pltpu.VMEM((1024,), jnp.int32),
            pltpu.SemaphoreType.DMA,
        ],
    )
    return fn(msg8, dbin, cnts)


_EBLK = 256
_NODE_BLK = 4000


def _edge_body(h_ref, wbig_ref, bbig_ref, msg_ref):
    m = jax.lax.dot_general(h_ref[...], wbig_ref[...], (((1,), (0,)), ((), ())),
                            preferred_element_type=jnp.float32)
    msg_ref[...] = jnp.maximum(m + bbig_ref[...], 0.0)


def _edge_mlp(h8, wbig, bbig):
    nblk = (_NPAD // 8) // _EBLK
    return pl.pallas_call(
        _edge_body,
        grid=(nblk,),
        in_specs=[
            pl.BlockSpec((_EBLK, 128), lambda i: (i, 0)),
            pl.BlockSpec((128, 256), lambda i: (0, 0)),
            pl.BlockSpec((1, 256), lambda i: (0, 0)),
        ],
        out_specs=pl.BlockSpec((_EBLK, 256), lambda i: (i, 0)),
        out_shape=jax.ShapeDtypeStruct((_NPAD // 8, 256), jnp.float32),
    )(h8, wbig, bbig)


def _node_body(x_ref, agg_ref, w2ax_ref, w2aa_ref, b2a_ref, w2b_ref, b2b_ref,
               w1a_ref, b1a_ref, comb_ref, np_ref):
    x = x_ref[...]
    agg = agg_ref[...]
    h = jax.lax.dot_general(x, w2ax_ref[...], (((1,), (0,)), ((), ())),
                            preferred_element_type=jnp.float32)
    h = h + jax.lax.dot_general(agg, w2aa_ref[...], (((1,), (0,)), ((), ())),
                                preferred_element_type=jnp.float32)
    h = jnp.maximum(h + b2a_ref[...], 0.0)
    comb = jax.lax.dot_general(h, w2b_ref[...], (((1,), (0,)), ((), ())),
                               preferred_element_type=jnp.float32)
    comb = jnp.maximum(comb + b2b_ref[...], 0.0)
    comb_ref[...] = comb
    x_next = jnp.concatenate([x[:, :11], comb], axis=1)
    npre = jax.lax.dot_general(x_next, w1a_ref[...], (((1,), (0,)), ((), ())),
                               preferred_element_type=jnp.float32)
    np_ref[...] = npre + b1a_ref[...]


def _node_mlp(x, agg, w2ax, w2aa, b2a, w2b, b2b, w1a, b1a):
    nblk = N_NODES // _NODE_BLK
    return pl.pallas_call(
        _node_body,
        grid=(nblk,),
        in_specs=[
            pl.BlockSpec((_NODE_BLK, 12), lambda i: (i, 0)),
            pl.BlockSpec((_NODE_BLK, 32), lambda i: (i, 0)),
            pl.BlockSpec((12, 16), lambda i: (0, 0)),
            pl.BlockSpec((32, 16), lambda i: (0, 0)),
            pl.BlockSpec((1, 16), lambda i: (0, 0)),
            pl.BlockSpec((16, 1), lambda i: (0, 0)),
            pl.BlockSpec((1, 1), lambda i: (0, 0)),
            pl.BlockSpec((12, 16), lambda i: (0, 0)),
            pl.BlockSpec((1, 16), lambda i: (0, 0)),
        ],
        out_specs=[
            pl.BlockSpec((_NODE_BLK, 1), lambda i: (i, 0)),
            pl.BlockSpec((_NODE_BLK, 16), lambda i: (i, 0)),
        ],
        out_shape=[
            jax.ShapeDtypeStruct((N_NODES, 1), jnp.float32),
            jax.ShapeDtypeStruct((N_NODES, 16), jnp.float32),
        ],
    )(x, agg, w2ax, w2aa, b2a, w2b, b2b, w1a, b1a)


def _pre_body(x_ref, w1a_ref, b1a_ref, np_ref):
    npre = jax.lax.dot_general(x_ref[...], w1a_ref[...], (((1,), (0,)), ((), ())),
                               preferred_element_type=jnp.float32)
    np_ref[...] = npre + b1a_ref[...]


def _node_pre(x, w1a, b1a):
    nblk = N_NODES // _NODE_BLK
    return pl.pallas_call(
        _pre_body,
        grid=(nblk,),
        in_specs=[
            pl.BlockSpec((_NODE_BLK, 12), lambda i: (i, 0)),
            pl.BlockSpec((12, 16), lambda i: (0, 0)),
            pl.BlockSpec((1, 16), lambda i: (0, 0)),
        ],
        out_specs=pl.BlockSpec((_NODE_BLK, 16), lambda i: (i, 0)),
        out_shape=jax.ShapeDtypeStruct((N_NODES, 16), jnp.float32),
    )(x, w1a, b1a)


def kernel(x, edge_index, edge_attr, W1a, b1a, W1b, b1b, W2a, b2a, W2b, b2b):
    src = edge_index[0].astype(jnp.int32)
    dst = edge_index[1].astype(jnp.int32)
    e0 = edge_attr[:, 0]
    e1 = edge_attr[:, 1]

    w1a_x = W1a[:12]
    wae = jnp.reshape(W1a[12:14], (32,))
    b1a_r = b1a.reshape(1, 16)
    wbig = jnp.kron(jnp.eye(8, dtype=jnp.float32), W1b)
    bbig = jnp.tile(b1b, 8).reshape(1, 256)
    w2a_x = W2a[:12]
    w2a_a = W2a[12:44]
    b2a_r = b2a.reshape(1, 16)
    b2b_r = b2b.reshape(1, 1)

    sbin, dbin, e0b, e1b, cnts = _sc_bin(src, dst, e0, e1)

    xc = x[:, :11]
    npre = _node_pre(x, w1a_x, b1a_r)
    x_cur = x
    for _ in range(3):
        h8 = _sc_gather(jnp.reshape(npre, (12500, 128)), sbin, e0b, e1b, wae)
        msg8 = _edge_mlp(h8, wbig, bbig)
        agg8 = _sc_segmax(msg8, dbin, cnts)
        agg = jnp.reshape(agg8[:N_NODES // 4], (N_NODES, 32))
        comb, npre = _node_mlp(x_cur, agg, w2a_x, w2a_a, b2a_r, W2b,
                               b2b_r, w1a_x, b1a_r)
        x_cur = jnp.concatenate([xc, comb], axis=1)
    return x_cur
